# Initial kernel scaffold; baseline (speedup 1.0000x reference)
import jax
import jax.numpy as jnp
from jax.experimental import pallas as pl


def kernel(x, params, edge_index_q0, edge_index_q1, edge_index_t0, edge_index_t1, edge_index_c0, edge_index_c1):
    raise NotImplementedError("write your pallas kernel here")


# speedup vs baseline: 32.2908x; 32.2908x over previous
import functools
import math

import jax
import jax.numpy as jnp
from jax import lax
from jax.experimental import pallas as pl
from jax.experimental.pallas import tpu as pltpu
from jax.experimental.pallas import tpu_sc as plsc

NDIM = 128
H = 8
DK = NDIM // H
N = 4096
NTYPES = 3
NMP = 6
E = 65536
EALL = NMP * E
SCALE = math.sqrt(DK)

_NC, _NS = 2, 16
_NW = _NC * _NS
_PERW = EALL // _NW
_GB = 128
_SB = 128
_EPC = EALL // _NC
_PERT = _EPC // _NS
_AROW = 3 * N
_ZROW = _AROW // _NS

_mesh = plsc.VectorSubcoreMesh(core_axis_name="c", subcore_axis_name="s",
                               num_cores=_NC, num_subcores=_NS)


def _mm_t(a, w):
    return lax.dot_general(a, w, (((1,), (1,)), ((), ())),
                           preferred_element_type=jnp.float32)


def _mm(a, w):
    return lax.dot_general(a, w, (((1,), (0,)), ((), ())),
                           preferred_element_type=jnp.float32)


def _ln(h, g, b):
    m = jnp.mean(h, axis=1, keepdims=True)
    v = jnp.mean((h - m) * (h - m), axis=1, keepdims=True)
    return (h - m) / jnp.sqrt(v + 1e-5) * g + b


def _qkv_body(x_ref, wq_ref, bq_ref, wk_ref, wv_ref, q_out, k_out, v_out):
    xt = x_ref[...]
    q_out[...] = _mm_t(xt, wq_ref[0]) + bq_ref[0]
    k_out[...] = _mm_t(xt, wk_ref[0])
    v_out[...] = _mm_t(xt, wv_ref[0])


_qkv_call = pl.pallas_call(
    _qkv_body,
    grid=(NTYPES,),
    in_specs=[
        pl.BlockSpec((N, NDIM), lambda t: (t, 0)),
        pl.BlockSpec((1, NDIM, NDIM), lambda t: (t, 0, 0)),
        pl.BlockSpec((1, 1, NDIM), lambda t: (t, 0, 0)),
        pl.BlockSpec((1, NDIM, NDIM), lambda t: (t, 0, 0)),
        pl.BlockSpec((1, NDIM, NDIM), lambda t: (t, 0, 0)),
    ],
    out_specs=[pl.BlockSpec((N, NDIM), lambda t: (t, 0))] * 3,
    out_shape=[jax.ShapeDtypeStruct((NTYPES * N, NDIM), jnp.float32)] * 3,
)


@functools.partial(
    pl.kernel,
    out_type=[jax.ShapeDtypeStruct((EALL, NDIM), jnp.float32)] * 3,
    mesh=_mesh,
    scratch_types=[
        pltpu.VMEM((_GB,), jnp.int32),
        pltpu.VMEM((_GB, NDIM), jnp.float32),
        pltpu.SemaphoreType.DMA,
    ],
)
def _sc_gather(ktab, qtab, vtab, srci, dsti, ks_out, qd_out, vs_out,
               idx_v, buf, sem):
    wid = lax.axis_index("s") * _NC + lax.axis_index("c")
    base = wid * _PERW

    def step(g, carry):
        b0 = base + g * _GB
        pltpu.sync_copy(srci.at[pl.ds(b0, _GB)], idx_v)
        pltpu.async_copy(ktab.at[idx_v], buf, sem).wait()
        pltpu.sync_copy(buf, ks_out.at[pl.ds(b0, _GB)])
        pltpu.async_copy(vtab.at[idx_v], buf, sem).wait()
        pltpu.sync_copy(buf, vs_out.at[pl.ds(b0, _GB)])
        pltpu.sync_copy(dsti.at[pl.ds(b0, _GB)], idx_v)
        pltpu.async_copy(qtab.at[idx_v], buf, sem).wait()
        pltpu.sync_copy(buf, qd_out.at[pl.ds(b0, _GB)])
        return carry

    lax.fori_loop(0, _PERW // _GB, step, 0)


_RC = 1024


def _score_body(g8_ref, ks_ref, qd_ref, vs_ref, wvc_out, z_out):
    p = ks_ref[...] * qd_ref[...]
    s8 = _mm(p, g8_ref[...])
    sc = jnp.exp(jnp.clip(s8 * (1.0 / SCALE), -5.0, 5.0))
    z_out[...] = jnp.concatenate(
        [sc, jnp.zeros((_RC, NDIM - H), jnp.float32)], axis=1)
    b = _mm_t(sc, g8_ref[...])
    wvc_out[...] = vs_ref[...] * b


_score_call = pl.pallas_call(
    _score_body,
    grid=(EALL // _RC,),
    in_specs=[
        pl.BlockSpec((NDIM, H), lambda i: (0, 0)),
        pl.BlockSpec((_RC, NDIM), lambda i: (i, 0)),
        pl.BlockSpec((_RC, NDIM), lambda i: (i, 0)),
        pl.BlockSpec((_RC, NDIM), lambda i: (i, 0)),
    ],
    out_specs=[
        pl.BlockSpec((_RC, NDIM), lambda i: (i, 0)),
        pl.BlockSpec((_RC, NDIM), lambda i: (i, 0)),
    ],
    out_shape=[
        jax.ShapeDtypeStruct((EALL, NDIM), jnp.float32),
        jax.ShapeDtypeStruct((EALL, NDIM), jnp.float32),
    ],
)


def _make_scatter(lanes):
    @functools.partial(
        pl.kernel,
        out_type=jax.ShapeDtypeStruct((NMP * N, lanes), jnp.float32),
        mesh=_mesh,
        scratch_types=[
            pltpu.VMEM((_SB,), jnp.int32),
            pltpu.VMEM((_SB, lanes), jnp.float32),
            pltpu.VMEM_SHARED((_AROW, lanes), jnp.float32),
        ],
    )
    def scatter(rows, dsti, zero, out, idx_v, buf, acc):
        cid = lax.axis_index("c")
        sid = lax.axis_index("s")
        r0 = sid * _ZROW
        pltpu.sync_copy(zero.at[pl.ds(r0, _ZROW)], acc.at[pl.ds(r0, _ZROW)])
        plsc.subcore_barrier()

        base = cid * _EPC + sid * _PERT

        def step(g, carry):
            b0 = base + g * _SB
            pltpu.sync_copy(dsti.at[pl.ds(b0, _SB)], idx_v)
            pltpu.sync_copy(rows.at[pl.ds(b0, _SB)], buf)
            pltpu.sync_copy(buf, acc.at[idx_v], add=True)
            return carry

        lax.fori_loop(0, _PERT // _SB, step, 0)
        plsc.subcore_barrier()

        go = cid * _AROW + r0
        pltpu.sync_copy(acc.at[pl.ds(r0, _ZROW)], out.at[pl.ds(go, _ZROW)])

    return scatter


_sc_scatter = _make_scatter(NDIM)


def _post_body(x_ref, wv_ref, z_ref, g2_ref, wo_ref, bo_ref, g1g_ref, g1b_ref,
               w1_ref, b1_ref, w2_ref, b2_ref, g2g_ref, g2b_ref,
               wa1_ref, ba1_ref, wa2_ref, h_out, s_out):
    zb = _mm(z_ref[...], g2_ref[...])
    o = wv_ref[...] / (zb + 1e-9)
    xt = x_ref[...]
    h = _ln(xt + _mm_t(o, wo_ref[0]) + bo_ref[0], g1g_ref[0], g1b_ref[0])
    f = jnp.maximum(_mm_t(h, w1_ref[0]) + b1_ref[0], 0.0)
    h2 = _ln(h + _mm_t(f, w2_ref[0]) + b2_ref[0], g2g_ref[0], g2b_ref[0])
    a = jnp.tanh(_mm_t(h2, wa1_ref[0]) + ba1_ref[0])
    s = _mm_t(a, wa2_ref[0])
    h_out[...] = h2
    s_out[...] = jnp.broadcast_to(jnp.mean(s), (1, 1, NDIM))


def _w3(shape):
    return pl.BlockSpec((1,) + shape, lambda m: (m // 2,) + (0,) * len(shape))


_post_call = pl.pallas_call(
    _post_body,
    grid=(NMP,),
    in_specs=[
        pl.BlockSpec((N, NDIM), lambda m: (m // 2, 0)),
        pl.BlockSpec((N, NDIM), lambda m: (m, 0)),
        pl.BlockSpec((N, NDIM), lambda m: (m, 0)),
        pl.BlockSpec((NDIM, NDIM), lambda m: (0, 0)),
        _w3((NDIM, NDIM)),
        _w3((1, NDIM)),
        _w3((1, NDIM)),
        _w3((1, NDIM)),
        _w3((4 * NDIM, NDIM)),
        _w3((1, 4 * NDIM)),
        _w3((NDIM, 4 * NDIM)),
        _w3((1, NDIM)),
        _w3((1, NDIM)),
        _w3((1, NDIM)),
        _w3((NDIM, NDIM)),
        _w3((1, NDIM)),
        _w3((1, NDIM)),
    ],
    out_specs=[
        pl.BlockSpec((N, NDIM), lambda m: (m, 0)),
        pl.BlockSpec((1, 1, NDIM), lambda m: (m, 0, 0)),
    ],
    out_shape=[
        jax.ShapeDtypeStruct((NMP * N, NDIM), jnp.float32),
        jax.ShapeDtypeStruct((NMP, 1, NDIM), jnp.float32),
    ],
)


def _comb_body(h_ref, s_ref, o_ref):
    m = s_ref[...][:, 0, :]
    mx = jnp.max(m, axis=0, keepdims=True)
    e = jnp.exp(m - mx)
    w = e / jnp.sum(e, axis=0, keepdims=True)
    o_ref[...] = h_ref[0:N, :] * w[0:1, :] + h_ref[N:2 * N, :] * w[1:2, :]


_comb_call = pl.pallas_call(
    _comb_body,
    grid=(NTYPES,),
    in_specs=[
        pl.BlockSpec((2 * N, NDIM), lambda t: (t, 0)),
        pl.BlockSpec((2, 1, NDIM), lambda t: (t, 0, 0)),
    ],
    out_specs=pl.BlockSpec((N, NDIM), lambda t: (t, 0)),
    out_shape=jax.ShapeDtypeStruct((NTYPES * N, NDIM), jnp.float32),
)


def kernel(x, params, edge_index_q0, edge_index_q1, edge_index_t0,
           edge_index_t1, edge_index_c0, edge_index_c1):
    edges = [edge_index_q0, edge_index_q1, edge_index_t0,
             edge_index_t1, edge_index_c0, edge_index_c1]

    def stk(name):
        return jnp.stack([params[t][name] for t in ('q', 't', 'c')])

    def stk1(name):
        return stk(name).reshape(NTYPES, 1, -1)

    q_all, k_all, v_all = _qkv_call(x, stk('Wq'), stk1('bq'), stk('Wk'),
                                    stk('Wv'))

    src_tab = jnp.concatenate([edges[m][0] + (m // 2) * N for m in range(NMP)])
    dst_tab = jnp.concatenate([edges[m][1] + (m // 2) * N for m in range(NMP)])
    dst_acc = jnp.concatenate([edges[m][1] + (m % 3) * N for m in range(NMP)])

    ks, qd, vs = _sc_gather(k_all, q_all, v_all, src_tab, dst_tab)

    lane = jnp.arange(NDIM)
    g8 = (lane[:, None] // DK == jnp.arange(H)[None, :]).astype(jnp.float32)
    wvc, z16 = _score_call(g8, ks, qd, vs)

    zero_acc = jnp.zeros((_AROW, NDIM), jnp.float32)
    wv = _sc_scatter(wvc, dst_acc, zero_acc)
    z = _sc_scatter(z16, dst_acc, zero_acc)

    g2 = (lane[:, None] == lane[None, :] // DK).astype(jnp.float32)
    h_all, ssum = _post_call(
        x, wv, z, g2, stk('Wo'), stk1('bo'), stk1('ln1_g'), stk1('ln1_b'),
        stk('W1'), stk1('b1'), stk('W2'), stk1('b2'), stk1('ln2_g'),
        stk1('ln2_b'), stk('Wa1'), stk1('ba1'), stk('Wa2'))

    return _comb_call(h_all, ssum)


# speedup vs baseline: 39.3700x; 1.2192x over previous
import functools
import math

import jax
import jax.numpy as jnp
from jax import lax
from jax.experimental import pallas as pl
from jax.experimental.pallas import tpu as pltpu
from jax.experimental.pallas import tpu_sc as plsc

NDIM = 128
H = 8
DK = NDIM // H
N = 4096
NTYPES = 3
NMP = 6
E = 65536
EALL = NMP * E
SCALE = math.sqrt(DK)

_NC, _NS = 2, 16
_NW = _NC * _NS
_PERW = EALL // _NW
_GB = 128
_SB = 128
_EPC = EALL // _NC
_PERT = _EPC // _NS
_AROW = 3 * N
_ZROW = _AROW // _NS

_mesh = plsc.VectorSubcoreMesh(core_axis_name="c", subcore_axis_name="s",
                               num_cores=_NC, num_subcores=_NS)


def _mm_t(a, w):
    return lax.dot_general(a, w, (((1,), (1,)), ((), ())),
                           preferred_element_type=jnp.float32)


def _mm(a, w):
    return lax.dot_general(a, w, (((1,), (0,)), ((), ())),
                           preferred_element_type=jnp.float32)


def _ln(h, g, b):
    m = jnp.mean(h, axis=1, keepdims=True)
    v = jnp.mean((h - m) * (h - m), axis=1, keepdims=True)
    return (h - m) / jnp.sqrt(v + 1e-5) * g + b


def _qkv_body(x_ref, wq_ref, bq_ref, wk_ref, wv_ref, q_out, kv_out):
    xt = x_ref[...]
    q_out[...] = _mm_t(xt, wq_ref[0]) + bq_ref[0]
    kv_out[...] = jnp.concatenate(
        [_mm_t(xt, wk_ref[0]), _mm_t(xt, wv_ref[0])], axis=1)


_qkv_call = pl.pallas_call(
    _qkv_body,
    grid=(NTYPES,),
    in_specs=[
        pl.BlockSpec((N, NDIM), lambda t: (t, 0)),
        pl.BlockSpec((1, NDIM, NDIM), lambda t: (t, 0, 0)),
        pl.BlockSpec((1, 1, NDIM), lambda t: (t, 0, 0)),
        pl.BlockSpec((1, NDIM, NDIM), lambda t: (t, 0, 0)),
        pl.BlockSpec((1, NDIM, NDIM), lambda t: (t, 0, 0)),
    ],
    out_specs=[
        pl.BlockSpec((N, NDIM), lambda t: (t, 0)),
        pl.BlockSpec((N, 2 * NDIM), lambda t: (t, 0)),
    ],
    out_shape=[
        jax.ShapeDtypeStruct((NTYPES * N, NDIM), jnp.float32),
        jax.ShapeDtypeStruct((NTYPES * N, 2 * NDIM), jnp.float32),
    ],
)


_NBLK = _PERW // _GB


@functools.partial(
    pl.kernel,
    out_type=[
        jax.ShapeDtypeStruct((EALL, 2 * NDIM), jnp.float32),
        jax.ShapeDtypeStruct((EALL, NDIM), jnp.float32),
    ],
    mesh=_mesh,
    scratch_types=[
        pltpu.VMEM((_NBLK, _GB), jnp.int32),
        pltpu.VMEM((_NBLK, _GB), jnp.int32),
        pltpu.VMEM((_GB, 2 * NDIM), jnp.float32),
        pltpu.VMEM((_GB, 2 * NDIM), jnp.float32),
        pltpu.VMEM((_GB, NDIM), jnp.float32),
        pltpu.VMEM((_GB, NDIM), jnp.float32),
        pltpu.SemaphoreType.DMA,
        pltpu.SemaphoreType.DMA,
        pltpu.SemaphoreType.DMA,
        pltpu.SemaphoreType.DMA,
    ],
)
def _sc_gather(kvtab, qtab, src2d, dst2d, kvs_out, qd_out,
               srcs_v, dsts_v, kvb0, kvb1, qb0, qb1, skv0, skv1, sq0, sq1):
    wid = lax.axis_index("s") * _NC + lax.axis_index("c")
    base = wid * _PERW
    pltpu.sync_copy(src2d.at[pl.ds(wid * _NBLK, _NBLK)], srcs_v)
    pltpu.sync_copy(dst2d.at[pl.ds(wid * _NBLK, _NBLK)], dsts_v)

    def issue(g, kvb, qb, skv, sq):
        pltpu.async_copy(kvtab.at[srcs_v.at[g]], kvb, skv)
        pltpu.async_copy(qtab.at[dsts_v.at[g]], qb, sq)

    def drain(g, kvb, qb, skv, sq):
        b0 = base + g * _GB
        pltpu.make_async_copy(kvtab.at[srcs_v.at[g]], kvb, skv).wait()
        pltpu.sync_copy(kvb, kvs_out.at[pl.ds(b0, _GB)])
        pltpu.make_async_copy(qtab.at[dsts_v.at[g]], qb, sq).wait()
        pltpu.sync_copy(qb, qd_out.at[pl.ds(b0, _GB)])

    issue(0, kvb0, qb0, skv0, sq0)

    def body2(j, carry):
        g0 = 2 * j
        issue(g0 + 1, kvb1, qb1, skv1, sq1)
        drain(g0, kvb0, qb0, skv0, sq0)

        @pl.when(j < _NBLK // 2 - 1)
        def _():
            issue(g0 + 2, kvb0, qb0, skv0, sq0)

        drain(g0 + 1, kvb1, qb1, skv1, sq1)
        return carry

    lax.fori_loop(0, _NBLK // 2, body2, 0)


_RC = 1024


def _score_body(g8_ref, kvs_ref, qd_ref, wvc_out, z_out):
    ks_ref = kvs_ref.at[:, 0:NDIM]
    vs_ref = kvs_ref.at[:, NDIM:2 * NDIM]
    p = ks_ref[...] * qd_ref[...]
    s8 = _mm(p, g8_ref[...])
    sc = jnp.exp(jnp.clip(s8 * (1.0 / SCALE), -5.0, 5.0))
    z_out[...] = jnp.concatenate(
        [sc, jnp.zeros((_RC, NDIM - H), jnp.float32)], axis=1)
    b = _mm_t(sc, g8_ref[...])
    wvc_out[...] = vs_ref[...] * b


_score_call = pl.pallas_call(
    _score_body,
    grid=(EALL // _RC,),
    in_specs=[
        pl.BlockSpec((NDIM, H), lambda i: (0, 0)),
        pl.BlockSpec((_RC, 2 * NDIM), lambda i: (i, 0)),
        pl.BlockSpec((_RC, NDIM), lambda i: (i, 0)),
    ],
    out_specs=[
        pl.BlockSpec((_RC, NDIM), lambda i: (i, 0)),
        pl.BlockSpec((_RC, NDIM), lambda i: (i, 0)),
    ],
    out_shape=[
        jax.ShapeDtypeStruct((EALL, NDIM), jnp.float32),
        jax.ShapeDtypeStruct((EALL, NDIM), jnp.float32),
    ],
)


def _make_scatter(lanes):
    @functools.partial(
        pl.kernel,
        out_type=jax.ShapeDtypeStruct((NMP * N, lanes), jnp.float32),
        mesh=_mesh,
        scratch_types=[
            pltpu.VMEM((_SB,), jnp.int32),
            pltpu.VMEM((_SB, lanes), jnp.float32),
            pltpu.VMEM_SHARED((_AROW, lanes), jnp.float32),
        ],
    )
    def scatter(rows, dsti, zero, out, idx_v, buf, acc):
        cid = lax.axis_index("c")
        sid = lax.axis_index("s")
        r0 = sid * _ZROW
        pltpu.sync_copy(zero.at[pl.ds(r0, _ZROW)], acc.at[pl.ds(r0, _ZROW)])
        plsc.subcore_barrier()

        base = cid * _EPC + sid * _PERT

        def step(g, carry):
            b0 = base + g * _SB
            pltpu.sync_copy(dsti.at[pl.ds(b0, _SB)], idx_v)
            pltpu.sync_copy(rows.at[pl.ds(b0, _SB)], buf)
            pltpu.sync_copy(buf, acc.at[idx_v], add=True)
            return carry

        lax.fori_loop(0, _PERT // _SB, step, 0)
        plsc.subcore_barrier()

        go = cid * _AROW + r0
        pltpu.sync_copy(acc.at[pl.ds(r0, _ZROW)], out.at[pl.ds(go, _ZROW)])

    return scatter


_sc_scatter = _make_scatter(NDIM)


def _post_body(x_ref, wv_ref, z_ref, g2_ref, wo_ref, bo_ref, g1g_ref, g1b_ref,
               w1_ref, b1_ref, w2_ref, b2_ref, g2g_ref, g2b_ref,
               wa1_ref, ba1_ref, wa2_ref, h_out, s_out):
    zb = _mm(z_ref[...], g2_ref[...])
    o = wv_ref[...] / (zb + 1e-9)
    xt = x_ref[...]
    h = _ln(xt + _mm_t(o, wo_ref[0]) + bo_ref[0], g1g_ref[0], g1b_ref[0])
    f = jnp.maximum(_mm_t(h, w1_ref[0]) + b1_ref[0], 0.0)
    h2 = _ln(h + _mm_t(f, w2_ref[0]) + b2_ref[0], g2g_ref[0], g2b_ref[0])
    a = jnp.tanh(_mm_t(h2, wa1_ref[0]) + ba1_ref[0])
    s = _mm_t(a, wa2_ref[0])
    h_out[...] = h2
    s_out[...] = jnp.broadcast_to(jnp.mean(s), (1, 1, NDIM))


def _w3(shape):
    return pl.BlockSpec((1,) + shape, lambda m: (m // 2,) + (0,) * len(shape))


_post_call = pl.pallas_call(
    _post_body,
    grid=(NMP,),
    in_specs=[
        pl.BlockSpec((N, NDIM), lambda m: (m // 2, 0)),
        pl.BlockSpec((N, NDIM), lambda m: (m, 0)),
        pl.BlockSpec((N, NDIM), lambda m: (m, 0)),
        pl.BlockSpec((NDIM, NDIM), lambda m: (0, 0)),
        _w3((NDIM, NDIM)),
        _w3((1, NDIM)),
        _w3((1, NDIM)),
        _w3((1, NDIM)),
        _w3((4 * NDIM, NDIM)),
        _w3((1, 4 * NDIM)),
        _w3((NDIM, 4 * NDIM)),
        _w3((1, NDIM)),
        _w3((1, NDIM)),
        _w3((1, NDIM)),
        _w3((NDIM, NDIM)),
        _w3((1, NDIM)),
        _w3((1, NDIM)),
    ],
    out_specs=[
        pl.BlockSpec((N, NDIM), lambda m: (m, 0)),
        pl.BlockSpec((1, 1, NDIM), lambda m: (m, 0, 0)),
    ],
    out_shape=[
        jax.ShapeDtypeStruct((NMP * N, NDIM), jnp.float32),
        jax.ShapeDtypeStruct((NMP, 1, NDIM), jnp.float32),
    ],
)


def _comb_body(h_ref, s_ref, o_ref):
    m = s_ref[...][:, 0, :]
    mx = jnp.max(m, axis=0, keepdims=True)
    e = jnp.exp(m - mx)
    w = e / jnp.sum(e, axis=0, keepdims=True)
    o_ref[...] = h_ref[0:N, :] * w[0:1, :] + h_ref[N:2 * N, :] * w[1:2, :]


_comb_call = pl.pallas_call(
    _comb_body,
    grid=(NTYPES,),
    in_specs=[
        pl.BlockSpec((2 * N, NDIM), lambda t: (t, 0)),
        pl.BlockSpec((2, 1, NDIM), lambda t: (t, 0, 0)),
    ],
    out_specs=pl.BlockSpec((N, NDIM), lambda t: (t, 0)),
    out_shape=jax.ShapeDtypeStruct((NTYPES * N, NDIM), jnp.float32),
)


def kernel(x, params, edge_index_q0, edge_index_q1, edge_index_t0,
           edge_index_t1, edge_index_c0, edge_index_c1):
    edges = [edge_index_q0, edge_index_q1, edge_index_t0,
             edge_index_t1, edge_index_c0, edge_index_c1]

    def stk(name):
        return jnp.stack([params[t][name] for t in ('q', 't', 'c')])

    def stk1(name):
        return stk(name).reshape(NTYPES, 1, -1)

    q_all, kv_all = _qkv_call(x, stk('Wq'), stk1('bq'), stk('Wk'), stk('Wv'))

    src_tab = jnp.concatenate([edges[m][0] + (m // 2) * N for m in range(NMP)])
    dst_tab = jnp.concatenate([edges[m][1] + (m // 2) * N for m in range(NMP)])
    dst_acc = jnp.concatenate([edges[m][1] + (m % 3) * N for m in range(NMP)])

    kvs, qd = _sc_gather(kv_all, q_all,
                         src_tab.reshape(EALL // _GB, _GB),
                         dst_tab.reshape(EALL // _GB, _GB))

    lane = jnp.arange(NDIM)
    g8 = (lane[:, None] // DK == jnp.arange(H)[None, :]).astype(jnp.float32)
    wvc, z16 = _score_call(g8, kvs, qd)

    zero_acc = jnp.zeros((_AROW, NDIM), jnp.float32)
    wv = _sc_scatter(wvc, dst_acc, zero_acc)
    z = _sc_scatter(z16, dst_acc, zero_acc)

    g2 = (lane[:, None] == lane[None, :] // DK).astype(jnp.float32)
    h_all, ssum = _post_call(
        x, wv, z, g2, stk('Wo'), stk1('bo'), stk1('ln1_g'), stk1('ln1_b'),
        stk('W1'), stk1('b1'), stk('W2'), stk1('b2'), stk1('ln2_g'),
        stk1('ln2_b'), stk('Wa1'), stk1('ba1'), stk('Wa2'))

    return _comb_call(h_all, ssum)


# speedup vs baseline: 47.0448x; 1.1949x over previous
import functools
import math

import jax
import jax.numpy as jnp
from jax import lax
from jax.experimental import pallas as pl
from jax.experimental.pallas import tpu as pltpu
from jax.experimental.pallas import tpu_sc as plsc

NDIM = 128
H = 8
DK = NDIM // H
N = 4096
NTYPES = 3
NMP = 6
E = 65536
EALL = NMP * E
SCALE = math.sqrt(DK)

_NC, _NS = 2, 16
_NW = _NC * _NS
_PERW = EALL // _NW
_GB = 128
_SB = 96
_EPC = EALL // _NC
_PERT = _EPC // _NS
_AROW = 3 * N
_ZROW = _AROW // _NS

_mesh = plsc.VectorSubcoreMesh(core_axis_name="c", subcore_axis_name="s",
                               num_cores=_NC, num_subcores=_NS)


def _mm_t(a, w):
    return lax.dot_general(a, w, (((1,), (1,)), ((), ())),
                           preferred_element_type=jnp.float32)


def _mm(a, w):
    return lax.dot_general(a, w, (((1,), (0,)), ((), ())),
                           preferred_element_type=jnp.float32)


def _ln(h, g, b):
    m = jnp.mean(h, axis=1, keepdims=True)
    v = jnp.mean((h - m) * (h - m), axis=1, keepdims=True)
    return (h - m) / jnp.sqrt(v + 1e-5) * g + b


def _qkv_body(x_ref, wq_ref, bq_ref, wk_ref, wv_ref, q_out, kv_out):
    xt = x_ref[...]
    q_out[...] = _mm_t(xt, wq_ref[0]) + bq_ref[0]
    kv_out[...] = jnp.concatenate(
        [_mm_t(xt, wk_ref[0]), _mm_t(xt, wv_ref[0])], axis=1)


_qkv_call = pl.pallas_call(
    _qkv_body,
    grid=(NTYPES,),
    in_specs=[
        pl.BlockSpec((N, NDIM), lambda t: (t, 0)),
        pl.BlockSpec((1, NDIM, NDIM), lambda t: (t, 0, 0)),
        pl.BlockSpec((1, 1, NDIM), lambda t: (t, 0, 0)),
        pl.BlockSpec((1, NDIM, NDIM), lambda t: (t, 0, 0)),
        pl.BlockSpec((1, NDIM, NDIM), lambda t: (t, 0, 0)),
    ],
    out_specs=[
        pl.BlockSpec((N, NDIM), lambda t: (t, 0)),
        pl.BlockSpec((N, 2 * NDIM), lambda t: (t, 0)),
    ],
    out_shape=[
        jax.ShapeDtypeStruct((NTYPES * N, NDIM), jnp.float32),
        jax.ShapeDtypeStruct((NTYPES * N, 2 * NDIM), jnp.float32),
    ],
)


_NBLK = _PERW // _GB


@functools.partial(
    pl.kernel,
    out_type=[
        jax.ShapeDtypeStruct((EALL, 2 * NDIM), jnp.float32),
        jax.ShapeDtypeStruct((EALL, NDIM), jnp.float32),
    ],
    mesh=_mesh,
    scratch_types=[
        pltpu.VMEM((_NBLK, _GB), jnp.int32),
        pltpu.VMEM((_NBLK, _GB), jnp.int32),
        pltpu.VMEM((_GB, 2 * NDIM), jnp.float32),
        pltpu.VMEM((_GB, 2 * NDIM), jnp.float32),
        pltpu.VMEM((_GB, NDIM), jnp.float32),
        pltpu.VMEM((_GB, NDIM), jnp.float32),
        pltpu.SemaphoreType.DMA,
        pltpu.SemaphoreType.DMA,
        pltpu.SemaphoreType.DMA,
        pltpu.SemaphoreType.DMA,
    ],
)
def _sc_gather(kvtab, qtab, src2d, dst2d, kvs_out, qd_out,
               srcs_v, dsts_v, kvb0, kvb1, qb0, qb1, skv0, skv1, sq0, sq1):
    wid = lax.axis_index("s") * _NC + lax.axis_index("c")
    base = wid * _PERW
    pltpu.sync_copy(src2d.at[pl.ds(wid * _NBLK, _NBLK)], srcs_v)
    pltpu.sync_copy(dst2d.at[pl.ds(wid * _NBLK, _NBLK)], dsts_v)

    def issue(g, kvb, qb, skv, sq):
        pltpu.async_copy(kvtab.at[srcs_v.at[g]], kvb, skv)
        pltpu.async_copy(qtab.at[dsts_v.at[g]], qb, sq)

    def drain(g, kvb, qb, skv, sq):
        b0 = base + g * _GB
        pltpu.make_async_copy(kvtab.at[srcs_v.at[g]], kvb, skv).wait()
        pltpu.sync_copy(kvb, kvs_out.at[pl.ds(b0, _GB)])
        pltpu.make_async_copy(qtab.at[dsts_v.at[g]], qb, sq).wait()
        pltpu.sync_copy(qb, qd_out.at[pl.ds(b0, _GB)])

    issue(0, kvb0, qb0, skv0, sq0)

    def body2(j, carry):
        g0 = 2 * j
        issue(g0 + 1, kvb1, qb1, skv1, sq1)
        drain(g0, kvb0, qb0, skv0, sq0)

        @pl.when(j < _NBLK // 2 - 1)
        def _():
            issue(g0 + 2, kvb0, qb0, skv0, sq0)

        drain(g0 + 1, kvb1, qb1, skv1, sq1)
        return carry

    lax.fori_loop(0, _NBLK // 2, body2, 0)


_RC = 1024


def _score_body(g8_ref, kvs_ref, qd_ref, wvc_out, z_out):
    ks_ref = kvs_ref.at[:, 0:NDIM]
    vs_ref = kvs_ref.at[:, NDIM:2 * NDIM]
    p = ks_ref[...] * qd_ref[...]
    s8 = _mm(p, g8_ref[...])
    sc = jnp.exp(jnp.clip(s8 * (1.0 / SCALE), -5.0, 5.0))
    z_out[...] = jnp.concatenate(
        [sc, jnp.zeros((_RC, NDIM - H), jnp.float32)], axis=1)
    b = _mm_t(sc, g8_ref[...])
    wvc_out[...] = vs_ref[...] * b


_score_call = pl.pallas_call(
    _score_body,
    grid=(EALL // _RC,),
    in_specs=[
        pl.BlockSpec((NDIM, H), lambda i: (0, 0)),
        pl.BlockSpec((_RC, 2 * NDIM), lambda i: (i, 0)),
        pl.BlockSpec((_RC, NDIM), lambda i: (i, 0)),
    ],
    out_specs=[
        pl.BlockSpec((_RC, NDIM), lambda i: (i, 0)),
        pl.BlockSpec((_RC, NDIM), lambda i: (i, 0)),
    ],
    out_shape=[
        jax.ShapeDtypeStruct((EALL, NDIM), jnp.float32),
        jax.ShapeDtypeStruct((EALL, NDIM), jnp.float32),
    ],
)


_SBLK = _PERT // _SB


def _make_scatter(lanes):
    @functools.partial(
        pl.kernel,
        out_type=jax.ShapeDtypeStruct((NMP * N, lanes), jnp.float32),
        mesh=_mesh,
        scratch_types=[
            pltpu.VMEM((_SB,), jnp.int32),
            pltpu.VMEM((_SB,), jnp.int32),
            pltpu.VMEM((_SB, lanes), jnp.float32),
            pltpu.VMEM((_SB, lanes), jnp.float32),
            pltpu.VMEM_SHARED((_AROW, lanes), jnp.float32),
            pltpu.SemaphoreType.DMA,
            pltpu.SemaphoreType.DMA,
            pltpu.SemaphoreType.DMA,
            pltpu.SemaphoreType.DMA,
        ],
    )
    def scatter(rows, dsti, zero, out, idx0, idx1, buf0, buf1, acc,
                si0, si1, sr0, sr1):
        cid = lax.axis_index("c")
        sid = lax.axis_index("s")
        r0 = sid * _ZROW
        pltpu.sync_copy(zero.at[pl.ds(r0, _ZROW)], acc.at[pl.ds(r0, _ZROW)])
        plsc.subcore_barrier()

        base = cid * _EPC + sid * _PERT

        def load(g, idxb, buf, si, sr):
            b0 = base + g * _SB
            pltpu.async_copy(dsti.at[pl.ds(b0, _SB)], idxb, si)
            pltpu.async_copy(rows.at[pl.ds(b0, _SB)], buf, sr)

        def addto(g, idxb, buf, si, sr):
            b0 = base + g * _SB
            pltpu.make_async_copy(dsti.at[pl.ds(b0, _SB)], idxb, si).wait()
            pltpu.make_async_copy(rows.at[pl.ds(b0, _SB)], buf, sr).wait()
            pltpu.sync_copy(buf, acc.at[idxb], add=True)

        load(0, idx0, buf0, si0, sr0)

        def body2(j, carry):
            g0 = 2 * j
            load(g0 + 1, idx1, buf1, si1, sr1)
            addto(g0, idx0, buf0, si0, sr0)

            @pl.when(j < _SBLK // 2 - 1)
            def _():
                load(g0 + 2, idx0, buf0, si0, sr0)

            addto(g0 + 1, idx1, buf1, si1, sr1)
            return carry

        lax.fori_loop(0, _SBLK // 2, body2, 0)
        plsc.subcore_barrier()

        go = cid * _AROW + r0
        pltpu.sync_copy(acc.at[pl.ds(r0, _ZROW)], out.at[pl.ds(go, _ZROW)])

    return scatter


_sc_scatter = _make_scatter(NDIM)


def _post_body(x_ref, wv_ref, z_ref, g2_ref, wo_ref, bo_ref, g1g_ref, g1b_ref,
               w1_ref, b1_ref, w2_ref, b2_ref, g2g_ref, g2b_ref,
               wa1_ref, ba1_ref, wa2_ref, h_out, s_out):
    zb = _mm(z_ref[...], g2_ref[...])
    o = wv_ref[...] / (zb + 1e-9)
    xt = x_ref[...]
    h = _ln(xt + _mm_t(o, wo_ref[0]) + bo_ref[0], g1g_ref[0], g1b_ref[0])
    f = jnp.maximum(_mm_t(h, w1_ref[0]) + b1_ref[0], 0.0)
    h2 = _ln(h + _mm_t(f, w2_ref[0]) + b2_ref[0], g2g_ref[0], g2b_ref[0])
    a = jnp.tanh(_mm_t(h2, wa1_ref[0]) + ba1_ref[0])
    s = _mm_t(a, wa2_ref[0])
    h_out[...] = h2
    s_out[...] = jnp.broadcast_to(jnp.mean(s), (1, 1, NDIM))


def _w3(shape):
    return pl.BlockSpec((1,) + shape, lambda m: (m // 2,) + (0,) * len(shape))


_post_call = pl.pallas_call(
    _post_body,
    grid=(NMP,),
    in_specs=[
        pl.BlockSpec((N, NDIM), lambda m: (m // 2, 0)),
        pl.BlockSpec((N, NDIM), lambda m: (m, 0)),
        pl.BlockSpec((N, NDIM), lambda m: (m, 0)),
        pl.BlockSpec((NDIM, NDIM), lambda m: (0, 0)),
        _w3((NDIM, NDIM)),
        _w3((1, NDIM)),
        _w3((1, NDIM)),
        _w3((1, NDIM)),
        _w3((4 * NDIM, NDIM)),
        _w3((1, 4 * NDIM)),
        _w3((NDIM, 4 * NDIM)),
        _w3((1, NDIM)),
        _w3((1, NDIM)),
        _w3((1, NDIM)),
        _w3((NDIM, NDIM)),
        _w3((1, NDIM)),
        _w3((1, NDIM)),
    ],
    out_specs=[
        pl.BlockSpec((N, NDIM), lambda m: (m, 0)),
        pl.BlockSpec((1, 1, NDIM), lambda m: (m, 0, 0)),
    ],
    out_shape=[
        jax.ShapeDtypeStruct((NMP * N, NDIM), jnp.float32),
        jax.ShapeDtypeStruct((NMP, 1, NDIM), jnp.float32),
    ],
)


def _comb_body(h_ref, s_ref, o_ref):
    m = s_ref[...][:, 0, :]
    mx = jnp.max(m, axis=0, keepdims=True)
    e = jnp.exp(m - mx)
    w = e / jnp.sum(e, axis=0, keepdims=True)
    o_ref[...] = h_ref[0:N, :] * w[0:1, :] + h_ref[N:2 * N, :] * w[1:2, :]


_comb_call = pl.pallas_call(
    _comb_body,
    grid=(NTYPES,),
    in_specs=[
        pl.BlockSpec((2 * N, NDIM), lambda t: (t, 0)),
        pl.BlockSpec((2, 1, NDIM), lambda t: (t, 0, 0)),
    ],
    out_specs=pl.BlockSpec((N, NDIM), lambda t: (t, 0)),
    out_shape=jax.ShapeDtypeStruct((NTYPES * N, NDIM), jnp.float32),
)


def kernel(x, params, edge_index_q0, edge_index_q1, edge_index_t0,
           edge_index_t1, edge_index_c0, edge_index_c1):
    edges = [edge_index_q0, edge_index_q1, edge_index_t0,
             edge_index_t1, edge_index_c0, edge_index_c1]

    def stk(name):
        return jnp.stack([params[t][name] for t in ('q', 't', 'c')])

    def stk1(name):
        return stk(name).reshape(NTYPES, 1, -1)

    q_all, kv_all = _qkv_call(x, stk('Wq'), stk1('bq'), stk('Wk'), stk('Wv'))

    src_tab = jnp.concatenate([edges[m][0] + (m // 2) * N for m in range(NMP)])
    dst_tab = jnp.concatenate([edges[m][1] + (m // 2) * N for m in range(NMP)])
    dst_acc = jnp.concatenate([edges[m][1] + (m % 3) * N for m in range(NMP)])

    kvs, qd = _sc_gather(kv_all, q_all,
                         src_tab.reshape(EALL // _GB, _GB),
                         dst_tab.reshape(EALL // _GB, _GB))

    lane = jnp.arange(NDIM)
    g8 = (lane[:, None] // DK == jnp.arange(H)[None, :]).astype(jnp.float32)
    wvc, z16 = _score_call(g8, kvs, qd)

    zero_acc = jnp.zeros((_AROW, NDIM), jnp.float32)
    wv = _sc_scatter(wvc, dst_acc, zero_acc)
    z = _sc_scatter(z16, dst_acc, zero_acc)

    g2 = (lane[:, None] == lane[None, :] // DK).astype(jnp.float32)
    h_all, ssum = _post_call(
        x, wv, z, g2, stk('Wo'), stk1('bo'), stk1('ln1_g'), stk1('ln1_b'),
        stk('W1'), stk1('b1'), stk('W2'), stk1('b2'), stk1('ln2_g'),
        stk1('ln2_b'), stk('Wa1'), stk1('ba1'), stk('Wa2'))

    return _comb_call(h_all, ssum)


# speedup vs baseline: 50.5771x; 1.0751x over previous
import functools
import math

import jax
import jax.numpy as jnp
from jax import lax
from jax.experimental import pallas as pl
from jax.experimental.pallas import tpu as pltpu
from jax.experimental.pallas import tpu_sc as plsc

NDIM = 128
H = 8
DK = NDIM // H
N = 4096
NTYPES = 3
NMP = 6
E = 65536
EALL = NMP * E
SCALE = math.sqrt(DK)

_NC, _NS = 2, 16
_NW = _NC * _NS
_E3 = 2 * E
_PERW = _E3 // _NW
_GB = 128
_SB = 128
_EPC = _E3 // _NC
_PERT = _EPC // _NS
_AROW = N
_ZROW = _AROW // _NS

_mesh = plsc.VectorSubcoreMesh(core_axis_name="c", subcore_axis_name="s",
                               num_cores=_NC, num_subcores=_NS)


def _mm_t(a, w):
    return lax.dot_general(a, w, (((1,), (1,)), ((), ())),
                           preferred_element_type=jnp.float32)


def _mm(a, w):
    return lax.dot_general(a, w, (((1,), (0,)), ((), ())),
                           preferred_element_type=jnp.float32)


def _ln(h, g, b):
    m = jnp.mean(h, axis=1, keepdims=True)
    v = jnp.mean((h - m) * (h - m), axis=1, keepdims=True)
    return (h - m) / jnp.sqrt(v + 1e-5) * g + b


def _qkv_body(x_ref, wq_ref, bq_ref, wk_ref, wv_ref, q_out, kv_out):
    xt = x_ref[...]
    q_out[...] = _mm_t(xt, wq_ref[0]) + bq_ref[0]
    kv_out[...] = jnp.concatenate(
        [_mm_t(xt, wk_ref[0]), _mm_t(xt, wv_ref[0])], axis=1)


_qkv_call = pl.pallas_call(
    _qkv_body,
    grid=(NTYPES,),
    in_specs=[
        pl.BlockSpec((N, NDIM), lambda t: (t, 0)),
        pl.BlockSpec((1, NDIM, NDIM), lambda t: (t, 0, 0)),
        pl.BlockSpec((1, 1, NDIM), lambda t: (t, 0, 0)),
        pl.BlockSpec((1, NDIM, NDIM), lambda t: (t, 0, 0)),
        pl.BlockSpec((1, NDIM, NDIM), lambda t: (t, 0, 0)),
    ],
    out_specs=[
        pl.BlockSpec((N, NDIM), lambda t: (t, 0)),
        pl.BlockSpec((N, 2 * NDIM), lambda t: (t, 0)),
    ],
    out_shape=[
        jax.ShapeDtypeStruct((NTYPES * N, NDIM), jnp.float32),
        jax.ShapeDtypeStruct((NTYPES * N, 2 * NDIM), jnp.float32),
    ],
)


_NBLK = _PERW // _GB


@functools.partial(
    pl.kernel,
    out_type=[
        jax.ShapeDtypeStruct((_E3, 2 * NDIM), jnp.float32),
        jax.ShapeDtypeStruct((_E3, NDIM), jnp.float32),
    ],
    mesh=_mesh,
    scratch_types=[
        pltpu.VMEM((_NBLK, _GB), jnp.int32),
        pltpu.VMEM((_NBLK, _GB), jnp.int32),
        pltpu.VMEM((_GB, 2 * NDIM), jnp.float32),
        pltpu.VMEM((_GB, 2 * NDIM), jnp.float32),
        pltpu.VMEM((_GB, NDIM), jnp.float32),
        pltpu.VMEM((_GB, NDIM), jnp.float32),
        pltpu.SemaphoreType.DMA,
        pltpu.SemaphoreType.DMA,
        pltpu.SemaphoreType.DMA,
        pltpu.SemaphoreType.DMA,
    ],
)
def _sc_gather(kvtab, qtab, src2d, dst2d, kvs_out, qd_out,
               srcs_v, dsts_v, kvb0, kvb1, qb0, qb1, skv0, skv1, sq0, sq1):
    wid = lax.axis_index("s") * _NC + lax.axis_index("c")
    base = wid * _PERW
    pltpu.sync_copy(src2d.at[pl.ds(wid * _NBLK, _NBLK)], srcs_v)
    pltpu.sync_copy(dst2d.at[pl.ds(wid * _NBLK, _NBLK)], dsts_v)

    def issue(g, kvb, qb, skv, sq):
        pltpu.async_copy(kvtab.at[srcs_v.at[g]], kvb, skv)
        pltpu.async_copy(qtab.at[dsts_v.at[g]], qb, sq)

    def drain(g, kvb, qb, skv, sq):
        b0 = base + g * _GB
        pltpu.make_async_copy(kvtab.at[srcs_v.at[g]], kvb, skv).wait()
        pltpu.sync_copy(kvb, kvs_out.at[pl.ds(b0, _GB)])
        pltpu.make_async_copy(qtab.at[dsts_v.at[g]], qb, sq).wait()
        pltpu.sync_copy(qb, qd_out.at[pl.ds(b0, _GB)])

    issue(0, kvb0, qb0, skv0, sq0)

    def body2(j, carry):
        g0 = 2 * j
        issue(g0 + 1, kvb1, qb1, skv1, sq1)
        drain(g0, kvb0, qb0, skv0, sq0)

        @pl.when(j < _NBLK // 2 - 1)
        def _():
            issue(g0 + 2, kvb0, qb0, skv0, sq0)

        drain(g0 + 1, kvb1, qb1, skv1, sq1)
        return carry

    lax.fori_loop(0, _NBLK // 2, body2, 0)


_RC = 1024


def _score_body(g8_ref, kvs_ref, qd_ref, wvc_out, z_out):
    ks_ref = kvs_ref.at[:, 0:NDIM]
    vs_ref = kvs_ref.at[:, NDIM:2 * NDIM]
    p = ks_ref[...] * qd_ref[...]
    s8 = _mm(p, g8_ref[...])
    sc = jnp.exp(jnp.clip(s8 * (1.0 / SCALE), -5.0, 5.0))
    z_out[...] = jnp.concatenate(
        [sc, jnp.zeros((_RC, NDIM - H), jnp.float32)], axis=1)
    b = _mm_t(sc, g8_ref[...])
    wvc_out[...] = vs_ref[...] * b


_score_call = pl.pallas_call(
    _score_body,
    grid=(_E3 // _RC,),
    in_specs=[
        pl.BlockSpec((NDIM, H), lambda i: (0, 0)),
        pl.BlockSpec((_RC, 2 * NDIM), lambda i: (i, 0)),
        pl.BlockSpec((_RC, NDIM), lambda i: (i, 0)),
    ],
    out_specs=[
        pl.BlockSpec((_RC, NDIM), lambda i: (i, 0)),
        pl.BlockSpec((_RC, NDIM), lambda i: (i, 0)),
    ],
    out_shape=[
        jax.ShapeDtypeStruct((_E3, NDIM), jnp.float32),
        jax.ShapeDtypeStruct((_E3, NDIM), jnp.float32),
    ],
)


_SBLK = _PERT // _SB


def _make_scatter(lanes):
    @functools.partial(
        pl.kernel,
        out_type=jax.ShapeDtypeStruct((_NC * N, lanes), jnp.float32),
        mesh=_mesh,
        scratch_types=[
            pltpu.VMEM((_SB,), jnp.int32),
            pltpu.VMEM((_SB,), jnp.int32),
            pltpu.VMEM((_SB, lanes), jnp.float32),
            pltpu.VMEM((_SB, lanes), jnp.float32),
            pltpu.VMEM_SHARED((_AROW, lanes), jnp.float32),
            pltpu.SemaphoreType.DMA,
            pltpu.SemaphoreType.DMA,
            pltpu.SemaphoreType.DMA,
            pltpu.SemaphoreType.DMA,
        ],
    )
    def scatter(rows, dsti, zero, out, idx0, idx1, buf0, buf1, acc,
                si0, si1, sr0, sr1):
        cid = lax.axis_index("c")
        sid = lax.axis_index("s")
        r0 = sid * _ZROW
        pltpu.sync_copy(zero.at[pl.ds(r0, _ZROW)], acc.at[pl.ds(r0, _ZROW)])
        plsc.subcore_barrier()

        base = cid * _EPC + sid * _PERT

        def load(g, idxb, buf, si, sr):
            b0 = base + g * _SB
            pltpu.async_copy(dsti.at[pl.ds(b0, _SB)], idxb, si)
            pltpu.async_copy(rows.at[pl.ds(b0, _SB)], buf, sr)

        def addto(g, idxb, buf, si, sr):
            b0 = base + g * _SB
            pltpu.make_async_copy(dsti.at[pl.ds(b0, _SB)], idxb, si).wait()
            pltpu.make_async_copy(rows.at[pl.ds(b0, _SB)], buf, sr).wait()
            pltpu.sync_copy(buf, acc.at[idxb], add=True)

        load(0, idx0, buf0, si0, sr0)

        def body2(j, carry):
            g0 = 2 * j
            load(g0 + 1, idx1, buf1, si1, sr1)
            addto(g0, idx0, buf0, si0, sr0)

            @pl.when(j < _SBLK // 2 - 1)
            def _():
                load(g0 + 2, idx0, buf0, si0, sr0)

            addto(g0 + 1, idx1, buf1, si1, sr1)
            return carry

        lax.fori_loop(0, _SBLK // 2, body2, 0)
        plsc.subcore_barrier()

        go = cid * _AROW + r0
        pltpu.sync_copy(acc.at[pl.ds(r0, _ZROW)], out.at[pl.ds(go, _ZROW)])

    return scatter


_sc_scatter = _make_scatter(NDIM)


def _post_body(x_ref, wv_ref, z_ref, g2_ref, wo_ref, bo_ref, g1g_ref, g1b_ref,
               w1_ref, b1_ref, w2_ref, b2_ref, g2g_ref, g2b_ref,
               wa1_ref, ba1_ref, wa2_ref, h_out, s_out):
    zb = _mm(z_ref[...], g2_ref[...])
    o = wv_ref[...] / (zb + 1e-9)
    xt = x_ref[...]
    h = _ln(xt + _mm_t(o, wo_ref[0]) + bo_ref[0], g1g_ref[0], g1b_ref[0])
    f = jnp.maximum(_mm_t(h, w1_ref[0]) + b1_ref[0], 0.0)
    h2 = _ln(h + _mm_t(f, w2_ref[0]) + b2_ref[0], g2g_ref[0], g2b_ref[0])
    a = jnp.tanh(_mm_t(h2, wa1_ref[0]) + ba1_ref[0])
    s = _mm_t(a, wa2_ref[0])
    h_out[...] = h2
    s_out[...] = jnp.broadcast_to(jnp.mean(s), (1, 1, NDIM))


def _w3(shape):
    return pl.BlockSpec((1,) + shape, lambda m: (m // 2,) + (0,) * len(shape))


_post_call = pl.pallas_call(
    _post_body,
    grid=(NMP,),
    in_specs=[
        pl.BlockSpec((N, NDIM), lambda m: (m // 2, 0)),
        pl.BlockSpec((N, NDIM), lambda m: (m, 0)),
        pl.BlockSpec((N, NDIM), lambda m: (m, 0)),
        pl.BlockSpec((NDIM, NDIM), lambda m: (0, 0)),
        _w3((NDIM, NDIM)),
        _w3((1, NDIM)),
        _w3((1, NDIM)),
        _w3((1, NDIM)),
        _w3((4 * NDIM, NDIM)),
        _w3((1, 4 * NDIM)),
        _w3((NDIM, 4 * NDIM)),
        _w3((1, NDIM)),
        _w3((1, NDIM)),
        _w3((1, NDIM)),
        _w3((NDIM, NDIM)),
        _w3((1, NDIM)),
        _w3((1, NDIM)),
    ],
    out_specs=[
        pl.BlockSpec((N, NDIM), lambda m: (m, 0)),
        pl.BlockSpec((1, 1, NDIM), lambda m: (m, 0, 0)),
    ],
    out_shape=[
        jax.ShapeDtypeStruct((NMP * N, NDIM), jnp.float32),
        jax.ShapeDtypeStruct((NMP, 1, NDIM), jnp.float32),
    ],
)


def _comb_body(h_ref, s_ref, o_ref):
    m = s_ref[...][:, 0, :]
    mx = jnp.max(m, axis=0, keepdims=True)
    e = jnp.exp(m - mx)
    w = e / jnp.sum(e, axis=0, keepdims=True)
    o_ref[...] = h_ref[0:N, :] * w[0:1, :] + h_ref[N:2 * N, :] * w[1:2, :]


_comb_call = pl.pallas_call(
    _comb_body,
    grid=(NTYPES,),
    in_specs=[
        pl.BlockSpec((2 * N, NDIM), lambda t: (t, 0)),
        pl.BlockSpec((2, 1, NDIM), lambda t: (t, 0, 0)),
    ],
    out_specs=pl.BlockSpec((N, NDIM), lambda t: (t, 0)),
    out_shape=jax.ShapeDtypeStruct((NTYPES * N, NDIM), jnp.float32),
)


def kernel(x, params, edge_index_q0, edge_index_q1, edge_index_t0,
           edge_index_t1, edge_index_c0, edge_index_c1):
    edges = [edge_index_q0, edge_index_q1, edge_index_t0,
             edge_index_t1, edge_index_c0, edge_index_c1]

    def stk(name):
        return jnp.stack([params[t][name] for t in ('q', 't', 'c')])

    def stk1(name):
        return stk(name).reshape(NTYPES, 1, -1)

    q_all, kv_all = _qkv_call(x, stk('Wq'), stk1('bq'), stk('Wk'), stk('Wv'))

    lane = jnp.arange(NDIM)
    g8 = (lane[:, None] // DK == jnp.arange(H)[None, :]).astype(jnp.float32)
    zero_acc = jnp.zeros((_AROW, NDIM), jnp.float32)

    wv_parts, z_parts = [], []
    for t in range(NTYPES):
        e0, e1 = edges[2 * t], edges[2 * t + 1]
        src3 = jnp.concatenate([e0[0], e1[0]]) + t * N
        dst3 = jnp.concatenate([e0[1], e1[1]]) + t * N
        dst_acc3 = jnp.concatenate([e0[1], e1[1]])
        kvs, qd = _sc_gather(kv_all, q_all,
                             src3.reshape(_E3 // _GB, _GB),
                             dst3.reshape(_E3 // _GB, _GB))
        wvc, z16 = _score_call(g8, kvs, qd)
        wv_parts.append(_sc_scatter(wvc, dst_acc3, zero_acc))
        z_parts.append(_sc_scatter(z16, dst_acc3, zero_acc))

    wv = jnp.concatenate(wv_parts)
    z = jnp.concatenate(z_parts)

    g2 = (lane[:, None] == lane[None, :] // DK).astype(jnp.float32)
    h_all, ssum = _post_call(
        x, wv, z, g2, stk('Wo'), stk1('bo'), stk1('ln1_g'), stk1('ln1_b'),
        stk('W1'), stk1('b1'), stk('W2'), stk1('b2'), stk1('ln2_g'),
        stk1('ln2_b'), stk('Wa1'), stk1('ba1'), stk('Wa2'))

    return _comb_call(h_all, ssum)
